# baseline (speedup 1.0000x reference)
import functools

import jax
import jax.numpy as jnp
from jax import lax
from jax.experimental import pallas as pl
from jax.experimental.pallas import tpu as pltpu
from jax.experimental.pallas import tpu_sc as plsc

_B = 16384
_F = 64
_NC = 2
_NS = 16
_NW = _NC * _NS
_BPW = _B // _NW
_CHUNK = 128
_NCHUNK = _BPW // _CHUNK
_GROUPS = _BPW // 16


def _body(uc_ref, ic_ref, uf_ref, if_ref, wb_ref,
          out_ref,
          idx_u, idx_i, u_rows, i_rows, wb_v, out_v, acc_m, sem):
    wid = lax.axis_index("s") * _NC + lax.axis_index("c")

    pltpu.sync_copy(uc_ref.at[wid], idx_u)
    pltpu.sync_copy(ic_ref.at[wid], idx_i)
    pltpu.sync_copy(wb_ref, wb_v)

    copies = []
    for j in range(_NCHUNK):
        copies.append(pltpu.make_async_copy(
            uf_ref.at[idx_u.at[j]], u_rows.at[pl.ds(j * _CHUNK, _CHUNK), :],
            sem))
        copies[-1].start()
    for j in range(_NCHUNK):
        copies.append(pltpu.make_async_copy(
            if_ref.at[idx_i.at[j]], i_rows.at[pl.ds(j * _CHUNK, _CHUNK), :],
            sem))
        copies[-1].start()
    for cp in copies:
        cp.wait()

    w0 = wb_v[pl.ds(0, 16)]
    w1 = wb_v[pl.ds(16, 16)]
    w2 = wb_v[pl.ds(32, 16)]
    w3 = wb_v[pl.ds(48, 16)]
    b = wb_v[pl.ds(64, 16)][0]
    lane = lax.iota(jnp.int32, 16)

    def group(g, carry):
        base = g * 16
        for j in range(16):
            r = base + j
            acc = (u_rows[r, pl.ds(0, 16)] * i_rows[r, pl.ds(0, 16)] * w0
                   + u_rows[r, pl.ds(16, 16)] * i_rows[r, pl.ds(16, 16)] * w1
                   + u_rows[r, pl.ds(32, 16)] * i_rows[r, pl.ds(32, 16)] * w2
                   + u_rows[r, pl.ds(48, 16)] * i_rows[r, pl.ds(48, 16)] * w3)
            acc_m[j, pl.ds(0, 16)] = acc
        tot = jnp.full((16,), b, jnp.float32)
        for c in range(16):
            col = plsc.load_gather(acc_m, [lane, jnp.full((16,), c, jnp.int32)])
            tot = tot + col
        out_v[pl.ds(base, 16)] = jnp.maximum(tot, 0.0)
        return carry

    lax.fori_loop(0, _GROUPS, group, 0)

    pltpu.sync_copy(out_v, out_ref.at[pl.ds(wid * _BPW, _BPW)])


@functools.partial(jax.jit, static_argnames=())
def kernel(user_coordinates, item_coordinates, USER_factors, ITEM_factors,
           W, b):
    uc = user_coordinates.astype(jnp.int32).reshape(_NW, _NCHUNK, _CHUNK)
    ic = item_coordinates.astype(jnp.int32).reshape(_NW, _NCHUNK, _CHUNK)
    wb = jnp.concatenate([W.reshape(_F), b.reshape(1),
                          jnp.zeros((15,), jnp.float32)])

    mesh = plsc.VectorSubcoreMesh(core_axis_name="c", subcore_axis_name="s")
    run = pl.kernel(
        _body,
        mesh=mesh,
        compiler_params=pltpu.CompilerParams(
            needs_layout_passes=False, use_tc_tiling_on_sc=False),
        out_type=jax.ShapeDtypeStruct((_B,), jnp.float32),
        scratch_types=[
            pltpu.VMEM((_NCHUNK, _CHUNK), jnp.int32),
            pltpu.VMEM((_NCHUNK, _CHUNK), jnp.int32),
            pltpu.VMEM((_BPW, _F), jnp.float32),
            pltpu.VMEM((_BPW, _F), jnp.float32),
            pltpu.VMEM((80,), jnp.float32),
            pltpu.VMEM((_BPW,), jnp.float32),
            pltpu.VMEM((16, 17), jnp.float32),
            pltpu.SemaphoreType.DMA,
        ],
    )
    out = run(uc, ic, USER_factors, ITEM_factors, wb)
    return out.reshape(_B, 1)


# speedup vs baseline: 2.6260x; 2.6260x over previous
import functools

import jax
import jax.numpy as jnp
from jax import lax
from jax.experimental import pallas as pl
from jax.experimental.pallas import tpu as pltpu
from jax.experimental.pallas import tpu_sc as plsc

_B = 16384
_F = 64
_NC = 2
_NS = 16
_NW = _NC * _NS
_BPW = _B // _NW
_GROUPS = _BPW // 16
_NBUF = 4


def _body(uc_ref, ic_ref, ufT_ref, ifT_ref, wb_ref,
          out_ref,
          idx_u, idx_i, u_tile, i_tile, wb_v, out_v, sems):
    wid = lax.axis_index("s") * _NC + lax.axis_index("c")
    base = wid * _BPW

    pltpu.sync_copy(uc_ref.at[pl.ds(base, _BPW)], idx_u)
    pltpu.sync_copy(ic_ref.at[pl.ds(base, _BPW)], idx_i)
    pltpu.sync_copy(wb_ref, wb_v)

    w = [wb_v[pl.ds(c * 16, 16)] for c in range(4)]
    b = wb_v[pl.ds(64, 16)][0]
    lane = lax.iota(jnp.int32, 16)

    def fetch(ru, ri, slot):
        pltpu.make_async_copy(
            ufT_ref.at[:, pl.ds((ru // 128) * 128, 128)],
            u_tile.at[slot], sems.at[slot]).start()
        pltpu.make_async_copy(
            ifT_ref.at[:, pl.ds((ri // 128) * 128, 128)],
            i_tile.at[slot], sems.at[slot]).start()

    def wait_slot(slot):
        pltpu.make_async_copy(
            ufT_ref.at[:, pl.ds(0, 128)], u_tile.at[slot],
            sems.at[slot]).wait()
        pltpu.make_async_copy(
            ifT_ref.at[:, pl.ds(0, 128)], i_tile.at[slot],
            sems.at[slot]).wait()

    ru0 = idx_u[pl.ds(0, 16)]
    ri0 = idx_i[pl.ds(0, 16)]
    for s in range(_NBUF):
        fetch(ru0[s], ri0[s], s)

    def group(g, carry):
        j0 = g * 16
        ru_vec = idx_u[pl.ds(j0, 16)]
        ri_vec = idx_i[pl.ds(j0, 16)]
        nxt = jnp.minimum(j0 + 16, _BPW - 16)
        ru_nxt = idx_u[pl.ds(nxt, 16)]
        ri_nxt = idx_i[pl.ds(nxt, 16)]
        last = g == _GROUPS - 1
        res = jnp.zeros((16,), jnp.float32)
        for jj in range(16):
            slot = jj % _NBUF
            ru = ru_vec[jj]
            ri = ri_vec[jj]
            wait_slot(slot)
            cuv = jnp.full((16,), ru % 128, jnp.int32)
            civ = jnp.full((16,), ri % 128, jnp.int32)
            acc = jnp.zeros((16,), jnp.float32)
            for c in range(4):
                fl = c * 16 + lane
                uv = plsc.load_gather(u_tile.at[slot], [fl, cuv])
                iv = plsc.load_gather(i_tile.at[slot], [fl, civ])
                acc = acc + uv * iv * w[c]
            if jj < 16 - _NBUF:
                run, rin = ru_vec[jj + _NBUF], ri_vec[jj + _NBUF]
            else:
                run = jnp.where(last, ru, ru_nxt[jj + _NBUF - 16])
                rin = jnp.where(last, ri, ri_nxt[jj + _NBUF - 16])
            fetch(run, rin, slot)
            s = jnp.sum(acc) + b
            res = jnp.where(lane == jj, jnp.maximum(s, 0.0), res)
        out_v[pl.ds(j0, 16)] = res
        return carry

    lax.fori_loop(0, _GROUPS, group, 0)

    for s in range(_NBUF):
        wait_slot(s)

    pltpu.sync_copy(out_v, out_ref.at[pl.ds(base, _BPW)])


@jax.jit
def kernel(user_coordinates, item_coordinates, USER_factors, ITEM_factors,
           W, b):
    uc = user_coordinates.astype(jnp.int32)
    ic = item_coordinates.astype(jnp.int32)
    ufT = USER_factors.T
    ifT = ITEM_factors.T
    wb = jnp.concatenate([W.reshape(_F), b.reshape(1),
                          jnp.zeros((15,), jnp.float32)])

    mesh = plsc.VectorSubcoreMesh(core_axis_name="c", subcore_axis_name="s")
    run = pl.kernel(
        _body,
        mesh=mesh,
        compiler_params=pltpu.CompilerParams(needs_layout_passes=False),
        out_type=jax.ShapeDtypeStruct((_B,), jnp.float32),
        scratch_types=[
            pltpu.VMEM((_BPW,), jnp.int32),
            pltpu.VMEM((_BPW,), jnp.int32),
            pltpu.VMEM((_NBUF, _F, 128), jnp.float32),
            pltpu.VMEM((_NBUF, _F, 128), jnp.float32),
            pltpu.VMEM((80,), jnp.float32),
            pltpu.VMEM((_BPW,), jnp.float32),
            pltpu.SemaphoreType.DMA((_NBUF,)),
        ],
    )
    out = run(uc, ic, ufT, ifT, wb)
    return out.reshape(_B, 1)
